# Initial kernel scaffold; baseline (speedup 1.0000x reference)
import jax
import jax.numpy as jnp
from jax.experimental import pallas as pl


def kernel(batch_seqs, token_table, pos_table):
    raise NotImplementedError("write your pallas kernel here")


# speedup vs baseline: 1.2523x; 1.2523x over previous
import functools

import jax
import jax.numpy as jnp
from jax import lax
from jax.experimental import pallas as pl
from jax.experimental.pallas import tpu as pltpu
from jax.experimental.pallas import tpu_sc as plsc

BATCH = 4096
MAX_LEN = 200
EMB = 32
NC = 2
NS = 16
NW = NC * NS
TOTAL = BATCH * MAX_LEN
ROWS_PER_W = TOTAL // NW
CHUNK = 2 * MAX_LEN
NCHUNK = ROWS_PER_W // CHUNK
GSUB = 80
NG = CHUNK // GSUB


def _body(idx_hbm, table_hbm, pos2_hbm, out_hbm, idx_v, buf, pos_v, sem):
    wid = lax.axis_index("s") * NC + lax.axis_index("c")
    pltpu.sync_copy(pos2_hbm, pos_v)

    def chunk_body(c, carry):
        flat_base = wid * ROWS_PER_W + c * CHUNK
        pltpu.sync_copy(idx_hbm.at[pl.ds(flat_base, CHUNK)], idx_v)
        copies = [
            pltpu.async_copy(
                table_hbm.at[idx_v.at[pl.ds(j * GSUB, GSUB)]],
                buf.at[pl.ds(j * GSUB, GSUB)],
                sem,
            )
            for j in range(NG)
        ]
        for cp in copies:
            cp.wait()

        def add_row(i, acc):
            buf[i, pl.ds(0, 16)] = buf[i, pl.ds(0, 16)] + pos_v[i, pl.ds(0, 16)]
            buf[i, pl.ds(16, 16)] = buf[i, pl.ds(16, 16)] + pos_v[i, pl.ds(16, 16)]
            return acc

        lax.fori_loop(0, CHUNK, add_row, 0)
        pltpu.sync_copy(buf, out_hbm.at[pl.ds(flat_base, CHUNK)])
        return carry

    lax.fori_loop(0, NCHUNK, chunk_body, 0)


@jax.jit
def _run(idx2, token_table, pos2):
    mesh = plsc.VectorSubcoreMesh(core_axis_name="c", subcore_axis_name="s")
    k = functools.partial(
        pl.kernel,
        mesh=mesh,
        out_type=jax.ShapeDtypeStruct((TOTAL, EMB), jnp.float32),
        scratch_types=[
            pltpu.VMEM((CHUNK,), jnp.int32),
            pltpu.VMEM((CHUNK, EMB), jnp.float32),
            pltpu.VMEM((CHUNK, EMB), jnp.float32),
            pltpu.SemaphoreType.DMA,
        ],
        compiler_params=pltpu.CompilerParams(use_tc_tiling_on_sc=False),
    )(_body)
    return k(idx2, token_table, pos2)


def kernel(batch_seqs, token_table, pos_table):
    idx = batch_seqs.reshape(TOTAL)
    pos2 = jnp.concatenate([pos_table, pos_table], axis=0)
    out = _run(idx, token_table, pos2)
    return out.reshape(BATCH, MAX_LEN, EMB)


# speedup vs baseline: 1.2834x; 1.0249x over previous
import functools

import jax
import jax.numpy as jnp
from jax import lax
from jax.experimental import pallas as pl
from jax.experimental.pallas import tpu as pltpu
from jax.experimental.pallas import tpu_sc as plsc

BATCH = 4096
MAX_LEN = 200
EMB = 32
NC = 2
NS = 16
NW = NC * NS
TOTAL = BATCH * MAX_LEN
ROWS_PER_W = TOTAL // NW
CHUNK = 2 * MAX_LEN
NCHUNK = ROWS_PER_W // CHUNK
GSUB = 80
NG = CHUNK // GSUB


def _body(idx_hbm, table_hbm, pos2_hbm, out_hbm, idx_v, buf, outb, pos_v, sem):
    wid = lax.axis_index("s") * NC + lax.axis_index("c")
    pltpu.sync_copy(pos2_hbm, pos_v)

    def chunk_body(c, carry):
        flat_base = wid * ROWS_PER_W + c * CHUNK
        pltpu.sync_copy(idx_hbm.at[pl.ds(flat_base, CHUNK)], idx_v)
        copies = [
            pltpu.async_copy(
                table_hbm.at[idx_v.at[pl.ds(j * GSUB, GSUB)]],
                buf.at[pl.ds(j * GSUB, GSUB)],
                sem,
            )
            for j in range(NG)
        ]
        for cp in copies:
            cp.wait()

        def add_row(i, acc):
            for h in range(2):
                outb[pl.ds(i * EMB + h * 16, 16)] = (
                    buf[i, pl.ds(h * 16, 16)] + pos_v[i, pl.ds(h * 16, 16)]
                )
            return acc

        lax.fori_loop(0, CHUNK, add_row, 0)
        pltpu.sync_copy(outb, out_hbm.at[pl.ds(flat_base * EMB, CHUNK * EMB)])
        return carry

    lax.fori_loop(0, NCHUNK, chunk_body, 0)


@jax.jit
def _run(idx, token_table, pos2):
    mesh = plsc.VectorSubcoreMesh(core_axis_name="c", subcore_axis_name="s")
    k = functools.partial(
        pl.kernel,
        mesh=mesh,
        out_type=jax.ShapeDtypeStruct((TOTAL * EMB,), jnp.float32),
        scratch_types=[
            pltpu.VMEM((CHUNK,), jnp.int32),
            pltpu.VMEM((CHUNK, EMB), jnp.float32),
            pltpu.VMEM((CHUNK * EMB,), jnp.float32),
            pltpu.VMEM((CHUNK, EMB), jnp.float32),
            pltpu.SemaphoreType.DMA,
        ],
        compiler_params=pltpu.CompilerParams(use_tc_tiling_on_sc=False),
    )(_body)
    return k(idx, token_table, pos2)


def kernel(batch_seqs, token_table, pos_table):
    idx = batch_seqs.reshape(TOTAL)
    pos2 = jnp.concatenate([pos_table, pos_table], axis=0)
    out = _run(idx, token_table, pos2)
    return out.reshape(BATCH, MAX_LEN, EMB)
